# Initial kernel scaffold; baseline (speedup 1.0000x reference)
import jax
import jax.numpy as jnp
from jax.experimental import pallas as pl


def kernel(x, W1, b1, W2, b2):
    raise NotImplementedError("write your pallas kernel here")


# speedup vs baseline: 53.1503x; 53.1503x over previous
import functools

import jax
import jax.numpy as jnp
from jax.experimental import pallas as pl

K = 4


def _stats_kernel(x_ref, sum_ref, topk_ref):
    v = x_ref[...]
    total = jnp.sum(v, axis=(1, 2))
    sum_ref[...] = total[:, None]

    acc = jnp.zeros_like(total)
    k_rem = jnp.full_like(total, float(K))
    for _ in range(K):
        m = jnp.max(v, axis=(1, 2))
        eq = v == m[:, None, None]
        cnt = jnp.sum(eq.astype(jnp.float32), axis=(1, 2))
        take = jnp.minimum(cnt, k_rem)
        acc = acc + jnp.where(take > 0, m * take, 0.0)
        k_rem = k_rem - take
        v = jnp.where(eq, -jnp.inf, v)
    topk_ref[...] = acc[:, None]


def _gate_kernel(sum_ref, topk_ref, w1_ref, b1_ref, w2_ref, b2_ref, gate_ref,
                 *, inv_n):
    avg = sum_ref[...] * inv_n
    tk = topk_ref[...]

    def fc(v):
        h = jnp.dot(w1_ref[...], v, preferred_element_type=jnp.float32)
        h = jnp.maximum(h + b1_ref[...], 0.0)
        o = jnp.dot(w2_ref[...], h, preferred_element_type=jnp.float32)
        return o + b2_ref[...]

    score = fc(avg) + fc(tk)
    gate_ref[...] = jax.nn.sigmoid(score)


def _scale_kernel(x_ref, gate_ref, out_ref):
    out_ref[...] = x_ref[...] * gate_ref[...][:, :, None]


def kernel(x, W1, b1, W2, b2):
    b, c, d, h, w = x.shape
    n = d * h * w
    assert b == 1
    lanes = 128
    rows = n // lanes
    xr = x.reshape(c, rows, lanes)

    cb = 8
    grid = c // cb

    sums, topks = pl.pallas_call(
        _stats_kernel,
        grid=(grid,),
        in_specs=[pl.BlockSpec((cb, rows, lanes), lambda i: (i, 0, 0))],
        out_specs=[
            pl.BlockSpec((cb, 1), lambda i: (i, 0)),
            pl.BlockSpec((cb, 1), lambda i: (i, 0)),
        ],
        out_shape=[
            jax.ShapeDtypeStruct((c, 1), jnp.float32),
            jax.ShapeDtypeStruct((c, 1), jnp.float32),
        ],
    )(xr)

    gate = pl.pallas_call(
        functools.partial(_gate_kernel, inv_n=1.0 / n),
        out_shape=jax.ShapeDtypeStruct((c, 1), jnp.float32),
    )(sums, topks, W1, b1[:, None], W2, b2[:, None])

    y = pl.pallas_call(
        _scale_kernel,
        grid=(grid,),
        in_specs=[
            pl.BlockSpec((cb, rows, lanes), lambda i: (i, 0, 0)),
            pl.BlockSpec((cb, 1), lambda i: (i, 0)),
        ],
        out_specs=pl.BlockSpec((cb, rows, lanes), lambda i: (i, 0, 0)),
        out_shape=jax.ShapeDtypeStruct((c, rows, lanes), jnp.float32),
    )(xr, gate)

    out = gate.reshape(b, c, 1, 1, 1)
    return (y.reshape(b, c, d, h, w), out)


# speedup vs baseline: 56.5220x; 1.0634x over previous
import functools

import jax
import jax.numpy as jnp
from jax.experimental import pallas as pl
from jax.experimental.pallas import tpu as pltpu

K = 4


def _stats_kernel(x_ref, sum_ref, topk_ref):
    v = x_ref[...]
    total = jnp.sum(v, axis=(1, 2))
    sum_ref[...] = total[:, None]

    cb, r, lanes = v.shape
    g = r // 8

    def body(i, carry):
        a1, a2, a3, a4 = carry
        s = x_ref[:, pl.ds(i * 8, 8), :]
        t = jnp.maximum(a1, s); s = jnp.minimum(a1, s); a1 = t
        t = jnp.maximum(a2, s); s = jnp.minimum(a2, s); a2 = t
        t = jnp.maximum(a3, s); s = jnp.minimum(a3, s); a3 = t
        a4 = jnp.maximum(a4, s)
        return a1, a2, a3, a4

    neg = jnp.full((cb, 8, lanes), -jnp.inf, jnp.float32)
    a1, a2, a3, a4 = jax.lax.fori_loop(0, g, body, (neg, neg, neg, neg))
    cand = jnp.concatenate([a1, a2, a3, a4], axis=1)

    acc = jnp.zeros((cb,), jnp.float32)
    k_rem = jnp.full((cb,), float(K))
    for _ in range(K):
        m = jnp.max(cand, axis=(1, 2))
        eq = cand == m[:, None, None]
        cnt = jnp.sum(eq.astype(jnp.float32), axis=(1, 2))
        take = jnp.minimum(cnt, k_rem)
        acc = acc + jnp.where(take > 0, m * take, 0.0)
        k_rem = k_rem - take
        cand = jnp.where(eq, -jnp.inf, cand)
    topk_ref[...] = acc[:, None]


def _gate_kernel(sum_ref, topk_ref, w1_ref, b1_ref, w2_ref, b2_ref, gate_ref,
                 *, inv_n):
    avg = sum_ref[...] * inv_n
    tk = topk_ref[...]

    def fc(v):
        h = jnp.dot(w1_ref[...], v, preferred_element_type=jnp.float32)
        h = jnp.maximum(h + b1_ref[...], 0.0)
        o = jnp.dot(w2_ref[...], h, preferred_element_type=jnp.float32)
        return o + b2_ref[...]

    score = fc(avg) + fc(tk)
    gate_ref[...] = jax.nn.sigmoid(score)


def _scale_kernel(x_ref, gate_ref, out_ref):
    out_ref[...] = x_ref[...] * gate_ref[...][:, :, None]


def kernel(x, W1, b1, W2, b2):
    b, c, d, h, w = x.shape
    n = d * h * w
    assert b == 1
    lanes = 128
    rows = n // lanes
    xr = x.reshape(c, rows, lanes)

    cb = 8
    grid = c // cb

    sums, topks = pl.pallas_call(
        _stats_kernel,
        grid=(grid,),
        in_specs=[pl.BlockSpec((cb, rows, lanes), lambda i: (i, 0, 0))],
        out_specs=[
            pl.BlockSpec((cb, 1), lambda i: (i, 0)),
            pl.BlockSpec((cb, 1), lambda i: (i, 0)),
        ],
        out_shape=[
            jax.ShapeDtypeStruct((c, 1), jnp.float32),
            jax.ShapeDtypeStruct((c, 1), jnp.float32),
        ],
        compiler_params=pltpu.CompilerParams(
            dimension_semantics=("parallel",)),
    )(xr)

    gate = pl.pallas_call(
        functools.partial(_gate_kernel, inv_n=1.0 / n),
        out_shape=jax.ShapeDtypeStruct((c, 1), jnp.float32),
    )(sums, topks, W1, b1[:, None], W2, b2[:, None])

    y = pl.pallas_call(
        _scale_kernel,
        grid=(grid,),
        in_specs=[
            pl.BlockSpec((cb, rows, lanes), lambda i: (i, 0, 0)),
            pl.BlockSpec((cb, 1), lambda i: (i, 0)),
        ],
        out_specs=pl.BlockSpec((cb, rows, lanes), lambda i: (i, 0, 0)),
        out_shape=jax.ShapeDtypeStruct((c, rows, lanes), jnp.float32),
        compiler_params=pltpu.CompilerParams(
            dimension_semantics=("parallel",)),
    )(xr, gate)

    out = gate.reshape(b, c, 1, 1, 1)
    return (y.reshape(b, c, d, h, w), out)
